# baseline (speedup 1.0000x reference)
import functools

import jax
import jax.numpy as jnp
from jax import lax
from jax.experimental import pallas as pl
from jax.experimental.pallas import tpu as pltpu
from jax.experimental.pallas import tpu_sc as plsc

B = 4096
S = 16
D = 512
E = 1024
C = 40

NC = 2
NS = 16
NW = NC * NS
BPW = B // NW
CHUNK = 32
L = 16


def _sc_gather_body(nodes_hbm, neigh_hbm, feat_hbm, self_out, agg_out,
                    sidx_v, nidx_v, rows_v, selfrows_v, acc_v,
                    sem_self, sem_n):
    wid = lax.axis_index("s") * NC + lax.axis_index("c")
    base = pl.multiple_of(wid * BPW, BPW)

    pltpu.sync_copy(nodes_hbm.at[pl.ds(base, BPW)], sidx_v)
    pltpu.sync_copy(neigh_hbm.at[pl.ds(base, BPW), :], nidx_v)

    self_copy = pltpu.async_copy(feat_hbm.at[sidx_v], selfrows_v, sem_self)

    def row_body(i, _):
        pltpu.async_copy(feat_hbm.at[nidx_v.at[i]], rows_v, sem_n).wait()
        ir = lax.rem(i, CHUNK)
        for j in range(D // L):
            sl = pl.ds(j * L, L)
            acc = rows_v[0, sl]
            for r in range(1, S):
                acc = acc + rows_v[r, sl]
            acc_v[ir, sl] = acc * (1.0 / S)

        @pl.when(ir == CHUNK - 1)
        def _flush():
            off = pl.multiple_of(base + i - (CHUNK - 1), CHUNK)
            pltpu.sync_copy(acc_v, agg_out.at[pl.ds(off, CHUNK)])

        return 0

    lax.fori_loop(0, BPW, row_body, 0)

    self_copy.wait()
    pltpu.sync_copy(selfrows_v, self_out.at[pl.ds(base, BPW)])


@functools.cache
def _make_sc_gather():
    return pl.kernel(
        _sc_gather_body,
        out_type=[
            jax.ShapeDtypeStruct((B, D), jnp.float32),
            jax.ShapeDtypeStruct((B, D), jnp.float32),
        ],
        mesh=plsc.VectorSubcoreMesh(core_axis_name="c", subcore_axis_name="s",
                                    num_cores=NC, num_subcores=NS),
        scratch_types=[
            pltpu.VMEM((BPW,), jnp.int32),
            pltpu.VMEM((BPW, S), jnp.int32),
            pltpu.VMEM((S, D), jnp.float32),
            pltpu.VMEM((BPW, D), jnp.float32),
            pltpu.VMEM((CHUNK, D), jnp.float32),
            pltpu.SemaphoreType.DMA,
            pltpu.SemaphoreType.DMA,
        ],
    )


def _tc_body(self_ref, agg_ref, w1_ref, w2_ref, wcls_ref, out_ref):
    h = lax.dot_general(self_ref[...], w1_ref[...],
                        (((1,), (1,)), ((), ())),
                        preferred_element_type=jnp.float32)
    h = h + lax.dot_general(agg_ref[...], w2_ref[...],
                            (((1,), (1,)), ((), ())),
                            preferred_element_type=jnp.float32)
    h = jnp.maximum(h, 0.0)
    out_ref[...] = lax.dot_general(h, wcls_ref[...],
                                   (((1,), (1,)), ((), ())),
                                   preferred_element_type=jnp.float32)


def _tc_head(self_feats, agg, w1, w2, wcls, block_b=512):
    grid = (B // block_b,)
    return pl.pallas_call(
        _tc_body,
        grid=grid,
        in_specs=[
            pl.BlockSpec((block_b, D), lambda i: (i, 0)),
            pl.BlockSpec((block_b, D), lambda i: (i, 0)),
            pl.BlockSpec((E, D), lambda i: (0, 0)),
            pl.BlockSpec((E, D), lambda i: (0, 0)),
            pl.BlockSpec((C, E), lambda i: (0, 0)),
        ],
        out_specs=pl.BlockSpec((block_b, C), lambda i: (i, 0)),
        out_shape=jax.ShapeDtypeStruct((B, C), jnp.float32),
    )(self_feats, agg, w1, w2, wcls)


def kernel(nodes, neigh_idx, features, W_enc, weight):
    nodes = nodes.astype(jnp.int32)
    neigh_idx = neigh_idx.astype(jnp.int32)
    self_feats, agg = _make_sc_gather()(nodes, neigh_idx, features)
    w1 = W_enc[:, :D]
    w2 = W_enc[:, D:]
    return _tc_head(self_feats, agg, w1, w2, weight)
